# baseline (speedup 1.0000x reference)
import functools

import jax
import jax.numpy as jnp
from jax import lax
from jax.experimental import pallas as pl
from jax.experimental.pallas import tpu as pltpu
from jax.experimental.pallas import tpu_sc as plsc

_LANES = 16


def _build_kernel(S, B0, V, D, W):
    info = plsc.get_sparse_core_info()
    nc = info.num_cores
    nw = nc * info.num_subcores
    assert B0 % nw == 0 and W == B0 // nw
    mesh = plsc.VectorSubcoreMesh(core_axis_name="c", subcore_axis_name="s")

    @functools.partial(
        pl.kernel,
        mesh=mesh,
        out_type=jax.ShapeDtypeStruct((S, D, B0), jnp.float32),
        scratch_types=[
            pltpu.VMEM((S, W), jnp.int32),
            pltpu.VMEM((W, D), jnp.float32),
            pltpu.VMEM((D, W), jnp.float32),
            pltpu.SemaphoreType.DMA,
            pltpu.SemaphoreType.DMA,
        ],
        compiler_params=pltpu.CompilerParams(
            use_tc_tiling_on_sc=False, needs_layout_passes=False
        ),
    )
    def k(xp_hbm, table_hbm, out_hbm, idx_all, rows_v, blk_v, gsem, wsem):
        wid = lax.axis_index("s") * nc + lax.axis_index("c")
        base = wid * W
        pltpu.sync_copy(xp_hbm.at[:, pl.ds(base, W)], idx_all)

        def body(s, carry):
            pltpu.async_copy(table_hbm.at[idx_all.at[s]], rows_v, gsem).wait()
            def trans_d(d, c2):
                for q in range(W // _LANES):
                    ridx = lax.iota(jnp.int32, _LANES) + (q * _LANES)
                    cidx = jnp.zeros((_LANES,), jnp.int32) + d
                    vals = plsc.load_gather(rows_v, [ridx, cidx])
                    blk_v[d, pl.ds(q * _LANES, _LANES)] = vals
                return c2

            lax.fori_loop(0, D, trans_d, 0)
            pltpu.async_copy(
                blk_v, out_hbm.at[s, :, pl.ds(base, W)], wsem
            ).wait()
            return carry

        lax.fori_loop(0, S, body, 0)

    return k


def kernel(x, emb_weight):
    B0, S = x.shape
    V, D = emb_weight.shape
    x_p = x.T
    out_p = _build_kernel(S, B0, V, D, B0 // 32)(x_p.astype(jnp.int32), emb_weight)
    return jnp.transpose(out_p, (2, 0, 1))


# speedup vs baseline: 1.1096x; 1.1096x over previous
import functools

import jax
import jax.numpy as jnp
from jax import lax
from jax.experimental import pallas as pl
from jax.experimental.pallas import tpu as pltpu
from jax.experimental.pallas import tpu_sc as plsc

_L = 16
_NBUF = 2


def _build_kernel(S, B0, V, D, W):
    info = plsc.get_sparse_core_info()
    nc = info.num_cores
    nw = nc * info.num_subcores
    assert B0 % nw == 0 and W == B0 // nw and S % _NBUF == 0
    mesh = plsc.VectorSubcoreMesh(core_axis_name="c", subcore_axis_name="s")

    @functools.partial(
        pl.kernel,
        mesh=mesh,
        out_type=jax.ShapeDtypeStruct((S, D, B0), jnp.float32),
        scratch_types=[
            pltpu.VMEM((S, W), jnp.int32),
            [pltpu.VMEM((W, D), jnp.float32) for _ in range(_NBUF)],
            [pltpu.VMEM((D, W), jnp.float32) for _ in range(_NBUF)],
            [pltpu.SemaphoreType.DMA for _ in range(_NBUF)],
            [pltpu.SemaphoreType.DMA for _ in range(_NBUF)],
        ],
        compiler_params=pltpu.CompilerParams(
            use_tc_tiling_on_sc=False, needs_layout_passes=False
        ),
    )
    def k(xp_hbm, table_hbm, out_hbm, idx_all, rows, blks, gsems, wsems):
        wid = lax.axis_index("s") * nc + lax.axis_index("c")
        base = wid * W
        pltpu.sync_copy(xp_hbm.at[:, pl.ds(base, W)], idx_all)

        def g_start(s, b):
            pltpu.async_copy(table_hbm.at[idx_all.at[s]], rows[b], gsems[b])

        def g_wait(s, b):
            pltpu.make_async_copy(
                table_hbm.at[idx_all.at[s]], rows[b], gsems[b]
            ).wait()

        def w_start(s, b):
            pltpu.async_copy(blks[b], out_hbm.at[s, :, pl.ds(base, W)], wsems[b])

        def w_wait(s, b):
            pltpu.make_async_copy(
                blks[b], out_hbm.at[s, :, pl.ds(base, W)], wsems[b]
            ).wait()

        def transpose(b):
            rv, bv = rows[b], blks[b]

            def td(d8, c):
                d0 = d8 * 8
                for dd in range(8):
                    d = d0 + dd
                    cvec = jnp.zeros((_L,), jnp.int32) + d
                    for q in range(W // _L):
                        rvec = lax.iota(jnp.int32, _L) + (q * _L)
                        vals = plsc.load_gather(rv, [rvec, cvec])
                        bv[d, pl.ds(q * _L, _L)] = vals
                return c

            lax.fori_loop(0, D // 8, td, 0)

        for b in range(_NBUF):
            g_start(b, b)
        for b in range(_NBUF):
            g_wait(b, b)
            transpose(b)
            w_start(b, b)
            g_start(b + _NBUF, b)

        def body(g, carry):
            for b in range(_NBUF):
                s = g * _NBUF + b
                g_wait(s, b)
                w_wait(s - _NBUF, b)
                transpose(b)
                w_start(s, b)
                g_start(s + _NBUF, b)
            return carry

        lax.fori_loop(1, S // _NBUF - 1, body, 0)

        for b in range(_NBUF):
            s = S - _NBUF + b
            g_wait(s, b)
            w_wait(s - _NBUF, b)
            transpose(b)
            w_start(s, b)
        for b in range(_NBUF):
            w_wait(S - _NBUF + b, b)

    return k


def kernel(x, emb_weight):
    B0, S = x.shape
    V, D = emb_weight.shape
    x_p = x.T
    out_p = _build_kernel(S, B0, V, D, B0 // 32)(x_p.astype(jnp.int32), emb_weight)
    return jnp.transpose(out_p, (2, 0, 1))


# speedup vs baseline: 1.8581x; 1.6746x over previous
import functools

import jax
import jax.numpy as jnp
from jax import lax
from jax.experimental import pallas as pl
from jax.experimental.pallas import tpu as pltpu
from jax.experimental.pallas import tpu_sc as plsc

_L = 16
_NBUF = 2
_SPG = 4


def _build_kernel(S, B0, V, D, W):
    info = plsc.get_sparse_core_info()
    nc = info.num_cores
    nw = nc * info.num_subcores
    assert B0 % nw == 0 and W == B0 // nw
    G = S // _SPG
    assert S % _SPG == 0 and G % _NBUF == 0
    mesh = plsc.VectorSubcoreMesh(core_axis_name="c", subcore_axis_name="s")

    @functools.partial(
        pl.kernel,
        mesh=mesh,
        out_type=jax.ShapeDtypeStruct((S, D, B0), jnp.float32),
        scratch_types=[
            pltpu.VMEM((S, W), jnp.int32),
            [pltpu.VMEM((_SPG * W, D), jnp.float32) for _ in range(_NBUF)],
            [pltpu.VMEM((D, W + 1), jnp.float32) for _ in range(_NBUF)],
            [pltpu.SemaphoreType.DMA for _ in range(_NBUF)],
            [pltpu.SemaphoreType.DMA for _ in range(_NBUF)],
        ],
        compiler_params=pltpu.CompilerParams(
            use_tc_tiling_on_sc=False, needs_layout_passes=False
        ),
    )
    def k(xp_hbm, table_hbm, out_hbm, idx_all, rows, blks, gsems, wsems):
        wid = lax.axis_index("s") * nc + lax.axis_index("c")
        base = wid * W
        pltpu.sync_copy(xp_hbm.at[:, pl.ds(base, W)], idx_all)

        def g_start(g, b):
            for j in range(_SPG):
                s = g * _SPG + j
                pltpu.async_copy(
                    table_hbm.at[idx_all.at[s]],
                    rows[b].at[pl.ds(j * W, W), :],
                    gsems[b],
                )

        def g_wait(g, b):
            for j in range(_SPG):
                s = g * _SPG + j
                pltpu.make_async_copy(
                    table_hbm.at[idx_all.at[s]],
                    rows[b].at[pl.ds(j * W, W), :],
                    gsems[b],
                ).wait()

        def w_start(s, wb):
            pltpu.async_copy(
                blks[wb].at[:, pl.ds(0, W)],
                out_hbm.at[s, :, pl.ds(base, W)],
                wsems[wb],
            )

        def w_wait(s, wb):
            pltpu.make_async_copy(
                blks[wb].at[:, pl.ds(0, W)],
                out_hbm.at[s, :, pl.ds(base, W)],
                wsems[wb],
            ).wait()

        def transpose(gb, j, wb):
            rv, bv = rows[gb], blks[wb]
            off = j * W

            def ti(i2, c):
                i0 = i2 * 2
                for di in range(2):
                    i = i0 + di
                    cvec = jnp.zeros((_L,), jnp.int32) + i
                    for p in range(D // _L):
                        rvec = lax.iota(jnp.int32, _L) + (p * _L)
                        vals = rv[off + i, pl.ds(p * _L, _L)]
                        plsc.store_scatter(bv, [rvec, cvec], vals)
                return c

            lax.fori_loop(0, W // 2, ti, 0)

        def process_group(g, gb, first):
            for j in range(_SPG):
                s = g * _SPG + j
                wb = j % 2
                if not first:
                    w_wait(s - 2, wb)
                elif j >= 2:
                    w_wait(s - 2, wb)
                transpose(gb, j, wb)
                w_start(s, wb)

        for b in range(_NBUF):
            g_start(b, b)
        g_wait(0, 0)
        process_group(0, 0, True)
        g_start(_NBUF, 0)

        def body(g, carry):
            for b in range(_NBUF):
                gg = g * _NBUF + b
                g_wait(gg, b)
                process_group(gg, b, False)
                g_start(gg + _NBUF, b)
            return carry

        g_wait(1, 1)
        process_group(1, 1, False)
        g_start(1 + _NBUF, 1)

        lax.fori_loop(1, G // _NBUF - 1, body, 0)

        for b in range(_NBUF):
            gg = G - _NBUF + b
            g_wait(gg, b)
            process_group(gg, b, False)
        for s in (S - 2, S - 1):
            w_wait(s, s % 2)

    return k


def kernel(x, emb_weight):
    B0, S = x.shape
    V, D = emb_weight.shape
    x_p = x.T
    out_p = _build_kernel(S, B0, V, D, B0 // 32)(x_p.astype(jnp.int32), emb_weight)
    return jnp.transpose(out_p, (2, 0, 1))
